# baseline (speedup 1.0000x reference)
import jax
import jax.numpy as jnp
from jax import lax
from jax.experimental import pallas as pl
from jax.experimental.pallas import tpu as pltpu
from jax.experimental.pallas import tpu_sc as plsc

NC, NS = 2, 16
NW = NC * NS
CHUNK = 128
NBUF = 7
GAHEAD = 6


def _make_plane_gather(batch: int, hist: int, dim: int):
    steps = hist

    def body(table_hbm, idst_hbm, out_hbm, idx_v, rows_v, gsem, osem):
        wid = lax.axis_index("s") * NC + lax.axis_index("c")
        base = wid * CHUNK

        pltpu.sync_copy(idst_hbm.at[:, pl.ds(base, CHUNK)], idx_v)

        def gather(i, buf):
            return pltpu.make_async_copy(
                table_hbm.at[idx_v.at[i]], rows_v.at[buf], gsem.at[i % 2]
            )

        def store(i, buf):
            return pltpu.make_async_copy(
                rows_v.at[buf], out_hbm.at[i, pl.ds(base, CHUNK)], osem
            )

        for j in range(min(GAHEAD, steps)):
            gather(j, j % NBUF).start()

        @pl.loop(0, steps)
        def _(i):
            j = i + GAHEAD
            jb = lax.rem(j, NBUF)

            @pl.when(jnp.logical_and(j >= NBUF, j < steps))
            def _():
                store(j - NBUF, jb).wait()

            @pl.when(j < steps)
            def _():
                gather(j, jb).start()

            ib = lax.rem(i, NBUF)
            gather(i, ib).wait()
            store(i, ib).start()

        for k in range(max(steps - NBUF, 0), steps):
            store(k, k % NBUF).wait()

    return pl.kernel(
        body,
        out_type=jax.ShapeDtypeStruct((hist, batch, dim), jnp.float32),
        mesh=plsc.VectorSubcoreMesh(core_axis_name="c", subcore_axis_name="s"),
        scratch_types=[
            pltpu.VMEM((hist, CHUNK), jnp.int32),
            pltpu.VMEM((NBUF, CHUNK, dim), jnp.float32),
            pltpu.SemaphoreType.DMA((2,)),
            pltpu.SemaphoreType.DMA,
        ],
    )


F_NBUF = 5
F_GAHEAD = 3


def _make_flat_gather(total: int, dim: int):
    ids_per_w = total // NW
    nchunk = ids_per_w // CHUNK
    nouter = nchunk // F_NBUF

    def body(table_hbm, idx_hbm, out_hbm, idx_v, rows_v, gsem, osem):
        wid = lax.axis_index("s") * NC + lax.axis_index("c")
        base = wid * ids_per_w

        def out_at(ci):
            return out_hbm.at[pl.ds(base + ci * CHUNK, CHUNK)]

        pltpu.sync_copy(idx_hbm.at[pl.ds(base, ids_per_w)], idx_v)

        def idx_at(ci):
            return idx_v.at[pl.ds(ci * CHUNK, CHUNK)]

        def start_gather(ci, buf):
            pltpu.async_copy(table_hbm.at[idx_at(ci)], rows_v.at[buf], gsem)

        for b in range(F_GAHEAD):
            start_gather(b, b)

        @pl.loop(0, nouter)
        def _(gi):
            for b in range(F_NBUF):
                ci = gi * F_NBUF + b
                nb = (b + F_GAHEAD) % F_NBUF

                def wait_store(pci=ci + F_GAHEAD - F_NBUF, pb=nb):
                    pltpu.make_async_copy(rows_v.at[pb], out_at(pci), osem).wait()

                if b < F_NBUF - F_GAHEAD:
                    @pl.when(gi > 0)
                    def _():
                        wait_store()
                else:
                    wait_store()

                if b < F_NBUF - F_GAHEAD:
                    start_gather(ci + F_GAHEAD, nb)
                else:
                    @pl.when(gi < nouter - 1)
                    def _():
                        start_gather(ci + F_GAHEAD, nb)

                pltpu.make_async_copy(
                    table_hbm.at[idx_at(ci)], rows_v.at[b], gsem
                ).wait()
                pltpu.async_copy(rows_v.at[b], out_at(ci), osem)

        for b in range(F_GAHEAD, F_NBUF):
            ci = (nouter - 1) * F_NBUF + b
            pltpu.make_async_copy(rows_v.at[b], out_at(ci), osem).wait()

    return pl.kernel(
        body,
        out_type=jax.ShapeDtypeStruct((total, dim), jnp.float32),
        mesh=plsc.VectorSubcoreMesh(core_axis_name="c", subcore_axis_name="s"),
        scratch_types=[
            pltpu.VMEM((ids_per_w,), jnp.int32),
            pltpu.VMEM((F_NBUF, CHUNK, dim), jnp.float32),
            pltpu.SemaphoreType.DMA,
            pltpu.SemaphoreType.DMA,
        ],
    )


def kernel(annotations, ids):
    batch, hist = ids.shape
    vocab, dim = annotations.shape
    ids = ids.astype(jnp.int32)

    if batch % (NW * CHUNK) == 0:
        ids_t = jnp.transpose(ids)
        out_t = _make_plane_gather(batch, hist, dim)(annotations, ids_t)
        return jnp.transpose(out_t, (1, 0, 2))

    total = batch * hist
    flat = ids.reshape(total)
    grain = NW * CHUNK * F_NBUF
    padded = (total + grain - 1) // grain * grain
    if padded != total:
        flat = jnp.pad(flat, (0, padded - total))
    out = _make_flat_gather(padded, dim)(annotations, flat)
    return out[:total].reshape(batch, hist, dim)


# speedup vs baseline: 1.0058x; 1.0058x over previous
import jax
import jax.numpy as jnp
from jax import lax
from jax.experimental import pallas as pl
from jax.experimental.pallas import tpu as pltpu
from jax.experimental.pallas import tpu_sc as plsc

NC, NS = 2, 16
NW = NC * NS
CHUNK = 128
NBUF = 7
GAHEAD = 6


def _make_plane_gather(batch: int, hist: int, dim: int):
    steps = hist

    pre = min(8, hist) if hist > 8 else hist

    def body(table_hbm, idst_hbm, out_hbm, idx_v, rows_v, gsem, isem, osem):
        wid = lax.axis_index("s") * NC + lax.axis_index("c")
        base = wid * CHUNK

        pltpu.sync_copy(
            idst_hbm.at[pl.ds(0, pre), pl.ds(base, CHUNK)],
            idx_v.at[pl.ds(0, pre)],
        )

        def rest_copy():
            return pltpu.make_async_copy(
                idst_hbm.at[pl.ds(pre, hist - pre), pl.ds(base, CHUNK)],
                idx_v.at[pl.ds(pre, hist - pre)],
                isem,
            )

        def gather(i, buf):
            return pltpu.make_async_copy(
                table_hbm.at[idx_v.at[i]], rows_v.at[buf], gsem.at[i % 2]
            )

        def store(i, buf):
            return pltpu.make_async_copy(
                rows_v.at[buf], out_hbm.at[i, pl.ds(base, CHUNK)], osem
            )

        for j in range(min(GAHEAD, steps, pre)):
            gather(j, j % NBUF).start()
        if hist > pre:
            rest_copy().start()

        @pl.loop(0, steps)
        def _(i):
            if hist > pre:
                @pl.when(i == 0)
                def _():
                    rest_copy().wait()

            j = i + GAHEAD
            jb = lax.rem(j, NBUF)

            @pl.when(jnp.logical_and(j >= NBUF, j < steps))
            def _():
                store(j - NBUF, jb).wait()

            @pl.when(j < steps)
            def _():
                gather(j, jb).start()

            ib = lax.rem(i, NBUF)
            gather(i, ib).wait()
            store(i, ib).start()

        for k in range(max(steps - NBUF, 0), steps):
            store(k, k % NBUF).wait()

    return pl.kernel(
        body,
        out_type=jax.ShapeDtypeStruct((hist, batch, dim), jnp.float32),
        mesh=plsc.VectorSubcoreMesh(core_axis_name="c", subcore_axis_name="s"),
        scratch_types=[
            pltpu.VMEM((hist, CHUNK), jnp.int32),
            pltpu.VMEM((NBUF, CHUNK, dim), jnp.float32),
            pltpu.SemaphoreType.DMA((2,)),
            pltpu.SemaphoreType.DMA,
            pltpu.SemaphoreType.DMA,
        ],
    )


F_NBUF = 5
F_GAHEAD = 3


def _make_flat_gather(total: int, dim: int):
    ids_per_w = total // NW
    nchunk = ids_per_w // CHUNK
    nouter = nchunk // F_NBUF

    def body(table_hbm, idx_hbm, out_hbm, idx_v, rows_v, gsem, osem):
        wid = lax.axis_index("s") * NC + lax.axis_index("c")
        base = wid * ids_per_w

        def out_at(ci):
            return out_hbm.at[pl.ds(base + ci * CHUNK, CHUNK)]

        pltpu.sync_copy(idx_hbm.at[pl.ds(base, ids_per_w)], idx_v)

        def idx_at(ci):
            return idx_v.at[pl.ds(ci * CHUNK, CHUNK)]

        def start_gather(ci, buf):
            pltpu.async_copy(table_hbm.at[idx_at(ci)], rows_v.at[buf], gsem)

        for b in range(F_GAHEAD):
            start_gather(b, b)

        @pl.loop(0, nouter)
        def _(gi):
            for b in range(F_NBUF):
                ci = gi * F_NBUF + b
                nb = (b + F_GAHEAD) % F_NBUF

                def wait_store(pci=ci + F_GAHEAD - F_NBUF, pb=nb):
                    pltpu.make_async_copy(rows_v.at[pb], out_at(pci), osem).wait()

                if b < F_NBUF - F_GAHEAD:
                    @pl.when(gi > 0)
                    def _():
                        wait_store()
                else:
                    wait_store()

                if b < F_NBUF - F_GAHEAD:
                    start_gather(ci + F_GAHEAD, nb)
                else:
                    @pl.when(gi < nouter - 1)
                    def _():
                        start_gather(ci + F_GAHEAD, nb)

                pltpu.make_async_copy(
                    table_hbm.at[idx_at(ci)], rows_v.at[b], gsem
                ).wait()
                pltpu.async_copy(rows_v.at[b], out_at(ci), osem)

        for b in range(F_GAHEAD, F_NBUF):
            ci = (nouter - 1) * F_NBUF + b
            pltpu.make_async_copy(rows_v.at[b], out_at(ci), osem).wait()

    return pl.kernel(
        body,
        out_type=jax.ShapeDtypeStruct((total, dim), jnp.float32),
        mesh=plsc.VectorSubcoreMesh(core_axis_name="c", subcore_axis_name="s"),
        scratch_types=[
            pltpu.VMEM((ids_per_w,), jnp.int32),
            pltpu.VMEM((F_NBUF, CHUNK, dim), jnp.float32),
            pltpu.SemaphoreType.DMA,
            pltpu.SemaphoreType.DMA,
        ],
    )


def kernel(annotations, ids):
    batch, hist = ids.shape
    vocab, dim = annotations.shape
    ids = ids.astype(jnp.int32)

    if batch % (NW * CHUNK) == 0:
        ids_t = jnp.transpose(ids)
        out_t = _make_plane_gather(batch, hist, dim)(annotations, ids_t)
        return jnp.transpose(out_t, (1, 0, 2))

    total = batch * hist
    flat = ids.reshape(total)
    grain = NW * CHUNK * F_NBUF
    padded = (total + grain - 1) // grain * grain
    if padded != total:
        flat = jnp.pad(flat, (0, padded - total))
    out = _make_flat_gather(padded, dim)(annotations, flat)
    return out[:total].reshape(batch, hist, dim)
